# baseline (speedup 1.0000x reference)
import functools

import numpy as np
import jax
import jax.numpy as jnp
from jax.experimental import pallas as pl
from jax.experimental.pallas import tpu as pltpu

_B, _T, _D = 1, 2048, 768
_H, _DH = 12, 64
_NIH, _DIDX = 4, 64
_TOPK = 512
_BQ = 256
_NBITS = 28


def _rope_tables():
    pos = np.arange(_T, dtype=np.float32)
    inv = 1.0 / (10000.0 ** (np.arange(0, _DH, 2, dtype=np.float32) / _DH))
    freqs = np.outer(pos, inv)
    emb = np.concatenate([freqs, freqs], axis=-1)
    cos = np.cos(emb)
    sin = np.sin(emb)
    return (np.tile(cos, (1, _H)).astype(np.float32),
            np.tile(sin, (1, _H)).astype(np.float32))


_COSF, _SINF = _rope_tables()


def _proj_kernel(x_ref, wq_ref, wk_ref, wv_ref, wg_ref, wvg_ref, wqi_ref,
                 wki_ref, ww_ref, cos_ref, sin_ref,
                 q_out, k_out, vvg_out, og_out, qi_out, ki_out, wgt_out):
    x = x_ref[...]
    cos = cos_ref[...]
    sin = sin_ref[...]
    half = _DH // 2
    lane = jax.lax.broadcasted_iota(jnp.int32, (x.shape[0], _H * _DH), 1)
    first_half = (lane % _DH) < half

    def rope(t):
        left = jnp.concatenate([t[:, half:], t[:, :half]], axis=1)
        right = jnp.concatenate([t[:, -half:], t[:, :-half]], axis=1)
        rot = jnp.where(first_half, -left, right)
        return t * cos + rot * sin

    dot = functools.partial(jnp.dot, preferred_element_type=jnp.float32)
    q_out[...] = rope(dot(x, wq_ref[...]))
    k_out[...] = rope(dot(x, wk_ref[...]))
    vvg_out[...] = dot(x, wv_ref[...]) * jax.nn.sigmoid(dot(x, wvg_ref[...]))
    og_out[...] = jax.nn.sigmoid(dot(x, wg_ref[...]))
    qi_out[...] = dot(x, wqi_ref[...])
    ki_out[...] = dot(x, wki_ref[...])
    wgt_out[...] = jax.nn.sigmoid(dot(x, ww_ref[...]))


def _round_bf16(x):
    u = jax.lax.bitcast_convert_type(x, jnp.uint32)
    r = (u + jnp.uint32(0x7FFF) + ((u >> jnp.uint32(16)) & jnp.uint32(1)))
    r = r & jnp.uint32(0xFFFF0000)
    return jax.lax.bitcast_convert_type(r, jnp.float32)


def _attn_kernel(qi_ref, wgt_ref, q_ref, og_ref, bias_ref, ki_ref, k_ref,
                 vvg_ref, wo_ref, out_ref, s_ref, m_ref, l_ref, acc_ref):
    i = pl.program_id(0)
    nt = (((1,), (1,)), ((), ()))
    nkb = _T // _BQ

    s_ref[...] = jnp.full((_BQ, _T), -1.0, jnp.float32)
    qihs = [_round_bf16(qi_ref[:, h * _DIDX:(h + 1) * _DIDX])
            for h in range(_NIH)]
    wgts = [_round_bf16(wgt_ref[:, h:h + 1]) for h in range(_NIH)]
    tri = (jax.lax.broadcasted_iota(jnp.int32, (_BQ, _BQ), 1) <=
           jax.lax.broadcasted_iota(jnp.int32, (_BQ, _BQ), 0))

    for j in range(nkb):
        @pl.when(j <= i)
        def _(j=j):
            kib = _round_bf16(ki_ref[j * _BQ:(j + 1) * _BQ, :])
            sj = jnp.zeros((_BQ, _BQ), jnp.float32)
            for h in range(_NIH):
                d = jax.lax.dot_general(qihs[h], kib, nt,
                                        preferred_element_type=jnp.float32)
                bias_h = jnp.broadcast_to(bias_ref[0:1, h:h + 1], (_BQ, _BQ))
                g = jax.nn.sigmoid(d * (1.0 / 8.0) + bias_h)
                sj = sj + wgts[h] * _round_bf16(g)
            sj = jnp.where((j < i) | tri, sj, -1.0)
            s_ref[:, j * _BQ:(j + 1) * _BQ] = sj

    s = s_ref[...]

    def body(_, carry):
        lo, hi = carry
        mid = (lo + hi) * 0.5
        cnt = jnp.sum((s >= mid).astype(jnp.float32), axis=1, keepdims=True)
        take = cnt >= float(_TOPK)
        return jnp.where(take, mid, lo), jnp.where(take, hi, mid)

    lo, _ = jax.lax.fori_loop(
        0, _NBITS, body,
        (jnp.zeros((_BQ, 1), jnp.float32), jnp.full((_BQ, 1), 4.0)))

    q = _round_bf16(q_ref[...])
    m_ref[...] = jnp.full((_BQ, 128), -1e30, jnp.float32)
    l_ref[...] = jnp.zeros((_BQ, 128), jnp.float32)
    acc_ref[...] = jnp.zeros((_BQ, _H * _DH), jnp.float32)

    for j in range(nkb):
        @pl.when(j <= i)
        def _(j=j):
            kj = _round_bf16(k_ref[j * _BQ:(j + 1) * _BQ, :])
            vj = vvg_ref[j * _BQ:(j + 1) * _BQ, :]
            sel = s_ref[:, j * _BQ:(j + 1) * _BQ] >= lo
            for h in range(_H):
                lg = jax.lax.dot_general(
                    q[:, h * _DH:(h + 1) * _DH],
                    kj[:, h * _DH:(h + 1) * _DH], nt,
                    preferred_element_type=jnp.float32) * 0.125
                lg = jnp.where(sel, lg, -1e30)
                m_old = m_ref[:, h:h + 1]
                m_new = jnp.maximum(m_old, jnp.max(lg, axis=1, keepdims=True))
                alpha = jnp.exp(m_old - m_new)
                p = jnp.where(sel, jnp.exp(lg - m_new), 0.0)
                l_ref[:, h:h + 1] = (l_ref[:, h:h + 1] * alpha +
                                     jnp.sum(p, axis=1, keepdims=True))
                hs = slice(h * _DH, (h + 1) * _DH)
                acc_ref[:, hs] = acc_ref[:, hs] * alpha + jnp.dot(
                    p, vj[:, hs], preferred_element_type=jnp.float32)
                m_ref[:, h:h + 1] = m_new

    outs = [acc_ref[:, h * _DH:(h + 1) * _DH] * (1.0 / l_ref[:, h:h + 1])
            for h in range(_H)]
    out = jnp.concatenate(outs, axis=1) * og_ref[...]
    out_ref[...] = jnp.dot(out, wo_ref[...], preferred_element_type=jnp.float32)


def kernel(x, Wq, Wk, Wv, Wo, Wg, Wvg, Wqi, Wki, Ww, bias_i):
    x2 = x.reshape(_T, _D)
    ww_pad = jnp.pad(Ww, ((0, 0), (0, 128 - _NIH)))
    cosf = jnp.asarray(_COSF)
    sinf = jnp.asarray(_SINF)

    nq = _T // _BQ
    row_spec = pl.BlockSpec((_BQ, _D), lambda i: (i, 0))
    full = lambda shape: pl.BlockSpec(shape, lambda i: (0,) * len(shape))

    q_r, k_r, vvg, og, qi, ki, wgt = pl.pallas_call(
        _proj_kernel,
        grid=(nq,),
        in_specs=[
            row_spec,
            full((_D, _H * _DH)), full((_D, _H * _DH)), full((_D, _H * _DH)),
            full((_D, _H * _DH)), full((_D, _H * _DH)),
            full((_D, _NIH * _DIDX)), full((_D, _DIDX)), full((_D, 128)),
            row_spec, row_spec,
        ],
        out_specs=[
            row_spec, row_spec, row_spec, row_spec,
            pl.BlockSpec((_BQ, _NIH * _DIDX), lambda i: (i, 0)),
            pl.BlockSpec((_BQ, _DIDX), lambda i: (i, 0)),
            pl.BlockSpec((_BQ, 128), lambda i: (i, 0)),
        ],
        out_shape=[
            jax.ShapeDtypeStruct((_T, _H * _DH), jnp.float32),
            jax.ShapeDtypeStruct((_T, _H * _DH), jnp.float32),
            jax.ShapeDtypeStruct((_T, _H * _DH), jnp.float32),
            jax.ShapeDtypeStruct((_T, _H * _DH), jnp.float32),
            jax.ShapeDtypeStruct((_T, _NIH * _DIDX), jnp.float32),
            jax.ShapeDtypeStruct((_T, _DIDX), jnp.float32),
            jax.ShapeDtypeStruct((_T, 128), jnp.float32),
        ],
    )(x2, Wq, Wk, Wv, Wg, Wvg, Wqi, Wki, ww_pad, cosf, sinf)

    bias_pad = jnp.pad(bias_i, (0, 128 - _NIH)).reshape(1, 128)

    out = pl.pallas_call(
        _attn_kernel,
        grid=(nq,),
        in_specs=[
            pl.BlockSpec((_BQ, _NIH * _DIDX), lambda i: (i, 0)),
            pl.BlockSpec((_BQ, 128), lambda i: (i, 0)),
            row_spec, row_spec,
            full((1, 128)), full((_T, _DIDX)),
            full((_T, _H * _DH)), full((_T, _H * _DH)),
            full((_D, _D)),
        ],
        out_specs=row_spec,
        out_shape=jax.ShapeDtypeStruct((_T, _D), jnp.float32),
        scratch_shapes=[
            pltpu.VMEM((_BQ, _T), jnp.float32),
            pltpu.VMEM((_BQ, 128), jnp.float32),
            pltpu.VMEM((_BQ, 128), jnp.float32),
            pltpu.VMEM((_BQ, _H * _DH), jnp.float32),
        ],
    )(qi, wgt, q_r, og, bias_pad, ki, k_r, vvg, Wo)

    return out.reshape(_B, _T, _D)


# speedup vs baseline: 2.1937x; 2.1937x over previous
import functools

import numpy as np
import jax
import jax.numpy as jnp
from jax.experimental import pallas as pl

_B, _T, _D = 1, 2048, 768
_H, _DH = 12, 64
_NIH, _DIDX = 4, 64
_TOPK = 512
_BQ = 256
_NBITS = 28


def _rope_tables():
    pos = np.arange(_T, dtype=np.float32)
    inv = 1.0 / (10000.0 ** (np.arange(0, _DH, 2, dtype=np.float32) / _DH))
    freqs = np.outer(pos, inv)
    emb = np.concatenate([freqs, freqs], axis=-1)
    cos = np.cos(emb)
    sin = np.sin(emb)
    return (np.tile(cos, (1, _H)).astype(np.float32),
            np.tile(sin, (1, _H)).astype(np.float32))


_COSF, _SINF = _rope_tables()


def _proj_kernel(x_ref, wq_ref, wk_ref, wv_ref, wg_ref, wvg_ref, wqi_ref,
                 wki_ref, ww_ref, cos_ref, sin_ref,
                 q_out, k_out, vvg_out, og_out, qi_out, ki_out, wgt_out):
    x = x_ref[...]
    cos = cos_ref[...]
    sin = sin_ref[...]
    half = _DH // 2
    lane = jax.lax.broadcasted_iota(jnp.int32, (x.shape[0], _H * _DH), 1)
    first_half = (lane % _DH) < half

    def rope(t):
        left = jnp.concatenate([t[:, half:], t[:, :half]], axis=1)
        right = jnp.concatenate([t[:, -half:], t[:, :-half]], axis=1)
        rot = jnp.where(first_half, -left, right)
        return t * cos + rot * sin

    dot = functools.partial(jnp.dot, preferred_element_type=jnp.float32)
    q_out[...] = rope(dot(x, wq_ref[...]))
    k_out[...] = rope(dot(x, wk_ref[...]))
    vvg_out[...] = dot(x, wv_ref[...]) * jax.nn.sigmoid(dot(x, wvg_ref[...]))
    og_out[...] = jax.nn.sigmoid(dot(x, wg_ref[...]))
    qi_out[...] = dot(x, wqi_ref[...])
    ki_out[...] = dot(x, wki_ref[...])
    wgt_out[...] = jax.nn.sigmoid(dot(x, ww_ref[...]))


def _round_bf16(x):
    u = jax.lax.bitcast_convert_type(x, jnp.uint32)
    r = (u + jnp.uint32(0x7FFF) + ((u >> jnp.uint32(16)) & jnp.uint32(1)))
    r = r & jnp.uint32(0xFFFF0000)
    return jax.lax.bitcast_convert_type(r, jnp.float32)


def _attn_kernel(qi_ref, wgt_ref, q_ref, og_ref, bias_ref, ki_ref, k_ref,
                 vvg_ref, wo_ref, out_ref):
    i = pl.program_id(0)
    nt = (((1,), (1,)), ((), ()))

    kib = _round_bf16(ki_ref[...])
    s = jnp.zeros((_BQ, _T), jnp.float32)
    for h in range(_NIH):
        qih = _round_bf16(qi_ref[:, h * _DIDX:(h + 1) * _DIDX])
        d = jax.lax.dot_general(qih, kib, nt,
                                preferred_element_type=jnp.float32)
        bias_h = jnp.broadcast_to(bias_ref[0:1, h:h + 1], (_BQ, _T))
        g = jax.nn.sigmoid(d * (1.0 / 8.0) + bias_h)
        s = s + _round_bf16(wgt_ref[:, h:h + 1]) * _round_bf16(g)
    row = i * _BQ + jax.lax.broadcasted_iota(jnp.int32, (_BQ, _T), 0)
    col = jax.lax.broadcasted_iota(jnp.int32, (_BQ, _T), 1)
    causal = col <= row
    s = jnp.where(causal, s, -1.0)

    def body(_, carry):
        lo, hi = carry
        mid = (lo + hi) * 0.5
        cnt = jnp.sum((s >= mid).astype(jnp.float32), axis=1, keepdims=True)
        take = cnt >= float(_TOPK)
        return jnp.where(take, mid, lo), jnp.where(take, hi, mid)

    lo, _ = jax.lax.fori_loop(
        0, _NBITS, body,
        (jnp.zeros((_BQ, 1), jnp.float32), jnp.full((_BQ, 1), 4.0)))
    allowed = s >= lo

    q = _round_bf16(q_ref[...])
    k = _round_bf16(k_ref[...])
    vvg = vvg_ref[...]
    outs = []
    for h in range(_H):
        qh = q[:, h * _DH:(h + 1) * _DH]
        kh = k[:, h * _DH:(h + 1) * _DH]
        lg = jax.lax.dot_general(qh, kh, nt,
                                 preferred_element_type=jnp.float32) * 0.125
        lg = jnp.where(allowed, lg, -1e30)
        m = jnp.max(lg, axis=1, keepdims=True)
        p = jnp.exp(lg - m)
        p = p * (1.0 / jnp.sum(p, axis=1, keepdims=True))
        outs.append(jnp.dot(p, vvg[:, h * _DH:(h + 1) * _DH],
                            preferred_element_type=jnp.float32))
    out = jnp.concatenate(outs, axis=1) * og_ref[...]
    out_ref[...] = jnp.dot(out, wo_ref[...], preferred_element_type=jnp.float32)


def kernel(x, Wq, Wk, Wv, Wo, Wg, Wvg, Wqi, Wki, Ww, bias_i):
    x2 = x.reshape(_T, _D)
    ww_pad = jnp.pad(Ww, ((0, 0), (0, 128 - _NIH)))
    cosf = jnp.asarray(_COSF)
    sinf = jnp.asarray(_SINF)

    nq = _T // _BQ
    row_spec = pl.BlockSpec((_BQ, _D), lambda i: (i, 0))
    full = lambda shape: pl.BlockSpec(shape, lambda i: (0,) * len(shape))

    q_r, k_r, vvg, og, qi, ki, wgt = pl.pallas_call(
        _proj_kernel,
        grid=(nq,),
        in_specs=[
            row_spec,
            full((_D, _H * _DH)), full((_D, _H * _DH)), full((_D, _H * _DH)),
            full((_D, _H * _DH)), full((_D, _H * _DH)),
            full((_D, _NIH * _DIDX)), full((_D, _DIDX)), full((_D, 128)),
            row_spec, row_spec,
        ],
        out_specs=[
            row_spec, row_spec, row_spec, row_spec,
            pl.BlockSpec((_BQ, _NIH * _DIDX), lambda i: (i, 0)),
            pl.BlockSpec((_BQ, _DIDX), lambda i: (i, 0)),
            pl.BlockSpec((_BQ, 128), lambda i: (i, 0)),
        ],
        out_shape=[
            jax.ShapeDtypeStruct((_T, _H * _DH), jnp.float32),
            jax.ShapeDtypeStruct((_T, _H * _DH), jnp.float32),
            jax.ShapeDtypeStruct((_T, _H * _DH), jnp.float32),
            jax.ShapeDtypeStruct((_T, _H * _DH), jnp.float32),
            jax.ShapeDtypeStruct((_T, _NIH * _DIDX), jnp.float32),
            jax.ShapeDtypeStruct((_T, _DIDX), jnp.float32),
            jax.ShapeDtypeStruct((_T, 128), jnp.float32),
        ],
    )(x2, Wq, Wk, Wv, Wg, Wvg, Wqi, Wki, ww_pad, cosf, sinf)

    bias_pad = jnp.pad(bias_i, (0, 128 - _NIH)).reshape(1, 128)

    out = pl.pallas_call(
        _attn_kernel,
        grid=(nq,),
        in_specs=[
            pl.BlockSpec((_BQ, _NIH * _DIDX), lambda i: (i, 0)),
            pl.BlockSpec((_BQ, 128), lambda i: (i, 0)),
            row_spec, row_spec,
            full((1, 128)), full((_T, _DIDX)),
            full((_T, _H * _DH)), full((_T, _H * _DH)),
            full((_D, _D)),
        ],
        out_specs=row_spec,
        out_shape=jax.ShapeDtypeStruct((_T, _D), jnp.float32),
    )(qi, wgt, q_r, og, bias_pad, ki, k_r, vvg, Wo)

    return out.reshape(_B, _T, _D)
